# baseline (speedup 1.0000x reference)
import functools

import jax
import jax.numpy as jnp
from jax import lax
from jax.experimental import pallas as pl
from jax.experimental.pallas import tpu as pltpu
from jax.experimental.pallas import tpu_sc as plsc

S = 3
N = 10000
E = 30000
D = 64
ED = 16

NC = 2
NS = 16
NW = NC * NS

CH = 96
NCH = 10
EPW = CH * NCH
EPAD = EPW * NW
NPT = N // NS

NOUT = 10240
W = NOUT // NS
GB = 96
HALF = EPAD // NC
NG = HALF // 16

BE = 768
BN = 1000

@functools.cache
def _sc_mesh():
    return plsc.VectorSubcoreMesh(
        core_axis_name="c", subcore_axis_name="s",
        num_cores=NC, num_subcores=NS)


@functools.cache
def _sc_gather_kernel():
    @functools.partial(
        pl.kernel,
        out_type=jax.ShapeDtypeStruct((EPAD, D), jnp.float32),
        mesh=_sc_mesh(),
        scratch_types=[
            pltpu.VMEM((NCH, CH), jnp.int32),
            pltpu.VMEM((EPW, D), jnp.float32),
            pltpu.SemaphoreType.DMA,
        ],
        compiler_params=pltpu.CompilerParams(use_tc_tiling_on_sc=False),
    )
    def _sc_gather(table_hbm, idx_hbm, out_hbm, idx_v, rows_v, sem):
        wid = lax.axis_index("s") * NC + lax.axis_index("c")
        pltpu.sync_copy(idx_hbm.at[wid], idx_v)
        descs = [
            pltpu.async_copy(
                table_hbm.at[idx_v.at[j]], rows_v.at[pl.ds(j * CH, CH)], sem)
            for j in range(NCH)
        ]
        for d in descs:
            d.wait()
        pltpu.sync_copy(rows_v, out_hbm.at[pl.ds(wid * EPW, EPW)])

    return _sc_gather


def _sc_gather(table, idx_r):
    return _sc_gather_kernel()(table, idx_r)


@functools.cache
def _sc_scatter_kernel():
    @functools.partial(
        pl.kernel,
        out_type=jax.ShapeDtypeStruct((NC, NOUT, D), jnp.float32),
        mesh=_sc_mesh(),
        scratch_types=[
            pltpu.VMEM((HALF,), jnp.int32),
            pltpu.VMEM((HALF + 2 * GB,), jnp.int32),
            pltpu.VMEM((HALF + 2 * GB,), jnp.int32),
            pltpu.VMEM((W + 8, D), jnp.float32),
            pltpu.VMEM((GB, D), jnp.float32),
            pltpu.VMEM((GB, D), jnp.float32),
            pltpu.SemaphoreType.DMA,
            pltpu.SemaphoreType.DMA,
        ],
        compiler_params=pltpu.CompilerParams(
            use_tc_tiling_on_sc=False, needs_layout_passes=False),
    )
    def _sc_scatter_k(msg_hbm, dst_hbm, zeros_hbm, out_hbm,
                      dst_v, pos_v, loc_v, acc_v, rows0_v, rows1_v,
                      sem0, sem1):
        c = lax.axis_index("c")
        s = lax.axis_index("s")
        lo = s * W
        pltpu.sync_copy(dst_hbm.at[c], dst_v)
        pltpu.sync_copy(zeros_hbm, acc_v.at[pl.ds(0, W)])
        iota = lax.iota(jnp.int32, 16)

        def scan_body(g, cur):
            v = dst_v[pl.ds(g * 16, 16)]
            l = v - lo
            m = (l >= 0) & (l < W)
            eid = c * HALF + g * 16 + iota
            plsc.store_compressed(pos_v.at[pl.ds(cur, 16)], eid, mask=m)
            plsc.store_compressed(loc_v.at[pl.ds(cur, 16)], l, mask=m)
            return cur + plsc.all_reduce_population_count(m)[0]

        cnt = lax.fori_loop(0, NG, scan_body, jnp.int32(0), unroll=4)
        for i in range(2 * GB // 16):
            pos_v[pl.ds(cnt + i * 16, 16)] = jnp.full((16,), EPAD - 1,
                                                      jnp.int32)
            loc_v[pl.ds(cnt + i * 16, 16)] = jnp.full((16,), W, jnp.int32)
        nb = (cnt + GB - 1) // GB
        nbe = jnp.maximum(((nb + 1) // 2) * 2, 2)

        bufs = ((rows0_v, sem0), (rows1_v, sem1))

        def start(b, buf, sem):
            pltpu.async_copy(
                msg_hbm.at[pos_v.at[pl.ds(b * GB, GB)]], buf, sem)

        def process(b, buf, sem):
            pltpu.make_async_copy(
                msg_hbm.at[pos_v.at[pl.ds(b * GB, GB)]], buf, sem).wait()
            for j in range(GB // 16):
                locs = loc_v[pl.ds(b * GB + j * 16, 16)]
                for t in range(16):
                    l = locs[t]
                    e = j * 16 + t
                    for k in range(D // 16):
                        sl = pl.ds(k * 16, 16)
                        acc_v[l, sl] = acc_v[l, sl] + buf[e, sl]

        start(0, rows0_v, sem0)
        start(1, rows1_v, sem1)

        def pair_body(i, _):
            for par in range(2):
                b = i * 2 + par
                buf, sem = bufs[par]
                process(b, buf, sem)

                @pl.when(b + 2 < nbe)
                def _():
                    start(b + 2, buf, sem)
            return 0

        lax.fori_loop(0, nbe // 2, pair_body, 0)
        pltpu.sync_copy(acc_v.at[pl.ds(0, W)], out_hbm.at[c, pl.ds(lo, W)])

    return _sc_scatter_k


def _sc_scatter(msg, dst_r, zeros_nd):
    return _sc_scatter_kernel()(msg, dst_r, zeros_nd)


def _combine_body(x_ref, ea_ref, w_ref, r_ref, o_ref):
    ea_b = ea_ref[...].astype(jnp.bfloat16)
    x_b = x_ref[...].astype(jnp.bfloat16)
    ew = jnp.dot(ea_b, w_ref[...],
                 preferred_element_type=jnp.float32).astype(jnp.bfloat16)
    xrep = jnp.dot(x_b, r_ref[...], preferred_element_type=jnp.float32)
    z = xrep * ew.astype(jnp.float32)
    width = D * D
    while width > D:
        width //= 2
        z = z[:, :width] + z[:, width:2 * width]
    o_ref[...] = z


def _tc_combine(xg, ea, wcat, rep):
    return pl.pallas_call(
        _combine_body,
        grid=(EPAD // BE,),
        in_specs=[
            pl.BlockSpec((BE, D), lambda i: (i, 0)),
            pl.BlockSpec((BE, ED), lambda i: (i, 0)),
            pl.BlockSpec((ED, D * D), lambda i: (0, 0)),
            pl.BlockSpec((D, D * D), lambda i: (0, 0)),
        ],
        out_specs=pl.BlockSpec((BE, D), lambda i: (i, 0)),
        out_shape=jax.ShapeDtypeStruct((EPAD, D), jnp.float32),
    )(xg, ea, wcat, rep)


def _gru_body(p_ref, h_ref, w_ref, b_ref, o_ref):
    m = jnp.maximum(p_ref[0] + p_ref[1] + b_ref[6:7, :], 0.0)
    h = h_ref[...]

    def mm(a, k):
        return jnp.dot(a, w_ref[k], preferred_element_type=jnp.float32)

    r = jax.nn.sigmoid(mm(m, 0) + b_ref[0:1, :] + mm(h, 3) + b_ref[3:4, :])
    z = jax.nn.sigmoid(mm(m, 1) + b_ref[1:2, :] + mm(h, 4) + b_ref[4:5, :])
    n = jnp.tanh(mm(m, 2) + b_ref[2:3, :] + r * (mm(h, 5) + b_ref[5:6, :]))
    o_ref[...] = (1.0 - z) * n + z * h


def _tc_gru(parts, h, w6, b7):
    return pl.pallas_call(
        _gru_body,
        grid=(N // BN,),
        in_specs=[
            pl.BlockSpec((NC, BN, D), lambda i: (0, i, 0)),
            pl.BlockSpec((BN, D), lambda i: (i, 0)),
            pl.BlockSpec((6, D, D), lambda i: (0, 0, 0)),
            pl.BlockSpec((7, D), lambda i: (0, 0)),
        ],
        out_specs=pl.BlockSpec((BN, D), lambda i: (i, 0)),
        out_shape=jax.ShapeDtypeStruct((N, D), jnp.float32),
    )(parts, h, w6, b7)


def kernel(x, edge_index, edge_attr, We, b_conv, Wih, Whh, bih, bhh):
    src = edge_index[0]
    dst = edge_index[1]
    pad = EPAD - E
    src_r = jnp.concatenate([src, jnp.zeros((pad,), jnp.int32)]).reshape(
        NW, NCH, CH)
    dst_r = jnp.concatenate([dst, jnp.zeros((pad,), jnp.int32)]).reshape(
        NC, HALF)
    ea_p = jnp.concatenate(
        [edge_attr, jnp.zeros((pad, ED), jnp.float32)], axis=0)
    zeros_nd = jnp.zeros((W, D), jnp.float32)

    wcat_all = We.transpose(0, 2, 1).astype(jnp.bfloat16)
    rep = jnp.kron(jnp.eye(D, dtype=jnp.float32),
                   jnp.ones((1, D), jnp.float32)).astype(jnp.bfloat16)
    w6_all = jnp.stack([
        Wih[:, 0:D], Wih[:, D:2 * D], Wih[:, 2 * D:],
        Whh[:, 0:D], Whh[:, D:2 * D], Whh[:, 2 * D:],
    ], axis=1).transpose(0, 1, 3, 2)
    b7_all = jnp.stack([
        bih[:, 0:D], bih[:, D:2 * D], bih[:, 2 * D:],
        bhh[:, 0:D], bhh[:, D:2 * D], bhh[:, 2 * D:],
        b_conv,
    ], axis=1)

    def step(i, h):
        wcat = lax.dynamic_index_in_dim(wcat_all, i, 0, keepdims=False)
        w6 = lax.dynamic_index_in_dim(w6_all, i, 0, keepdims=False)
        b7 = lax.dynamic_index_in_dim(b7_all, i, 0, keepdims=False)
        xg = _sc_gather(h, src_r)
        msg = _tc_combine(xg, ea_p, wcat, rep)
        parts = _sc_scatter(msg, dst_r, zeros_nd)
        return _tc_gru(parts, h, w6, b7)

    return lax.fori_loop(0, S, step, x)


# speedup vs baseline: 1.0693x; 1.0693x over previous
import functools

import jax
import jax.numpy as jnp
from jax import lax
from jax.experimental import pallas as pl
from jax.experimental.pallas import tpu as pltpu
from jax.experimental.pallas import tpu_sc as plsc

S = 3
N = 10000
E = 30000
D = 64
ED = 16

NC = 2
NS = 16
NW = NC * NS

CH = 96
NCH = 10
EPW = CH * NCH
EPAD = EPW * NW
NPT = N // NS

NOUT = 10240
W = NOUT // NS
GB = 96
HALF = EPAD // NC
NG = HALF // 16

BE = 768
BN = 1000

@functools.cache
def _sc_mesh():
    return plsc.VectorSubcoreMesh(
        core_axis_name="c", subcore_axis_name="s",
        num_cores=NC, num_subcores=NS)


@functools.cache
def _sc_gather_kernel():
    @functools.partial(
        pl.kernel,
        out_type=jax.ShapeDtypeStruct((EPAD, D), jnp.float32),
        mesh=_sc_mesh(),
        scratch_types=[
            pltpu.VMEM((NCH, CH), jnp.int32),
            pltpu.VMEM((EPW, D), jnp.float32),
            pltpu.SemaphoreType.DMA,
        ],
        compiler_params=pltpu.CompilerParams(use_tc_tiling_on_sc=False),
    )
    def _sc_gather(table_hbm, idx_hbm, out_hbm, idx_v, rows_v, sem):
        wid = lax.axis_index("s") * NC + lax.axis_index("c")
        pltpu.sync_copy(idx_hbm.at[wid], idx_v)
        descs = [
            pltpu.async_copy(
                table_hbm.at[idx_v.at[j]], rows_v.at[pl.ds(j * CH, CH)], sem)
            for j in range(NCH)
        ]
        for d in descs:
            d.wait()
        pltpu.sync_copy(rows_v, out_hbm.at[pl.ds(wid * EPW, EPW)])

    return _sc_gather


def _sc_gather(table, idx_r):
    return _sc_gather_kernel()(table, idx_r)


@functools.cache
def _sc_scatter_kernel():
    @functools.partial(
        pl.kernel,
        out_type=jax.ShapeDtypeStruct((NC, NOUT, D), jnp.float32),
        mesh=_sc_mesh(),
        scratch_types=[
            pltpu.VMEM((HALF,), jnp.int32),
            pltpu.VMEM((HALF + 2 * GB,), jnp.int32),
            pltpu.VMEM((HALF + 2 * GB,), jnp.int32),
            pltpu.VMEM((W + 8, D), jnp.float32),
            pltpu.VMEM((GB, D), jnp.float32),
            pltpu.SemaphoreType.DMA,
        ],
        compiler_params=pltpu.CompilerParams(
            use_tc_tiling_on_sc=False, needs_layout_passes=False),
    )
    def _sc_scatter_k(msg_hbm, dst_hbm, zeros_hbm, out_hbm,
                      dst_v, pos_v, loc_v, acc_v, rows0_v, sem0):
        c = lax.axis_index("c")
        s = lax.axis_index("s")
        lo = s * W
        pltpu.sync_copy(dst_hbm.at[c], dst_v)
        pltpu.sync_copy(zeros_hbm, acc_v.at[pl.ds(0, W)])
        iota = lax.iota(jnp.int32, 16)

        def scan_body(g, cur):
            v = dst_v[pl.ds(g * 16, 16)]
            l = v - lo
            m = (l >= 0) & (l < W)
            eid = c * HALF + g * 16 + iota
            plsc.store_compressed(pos_v.at[pl.ds(cur, 16)], eid, mask=m)
            plsc.store_compressed(loc_v.at[pl.ds(cur, 16)], l, mask=m)
            return cur + plsc.all_reduce_population_count(m)[0]

        cnt = lax.fori_loop(0, NG, scan_body, jnp.int32(0), unroll=4)
        for i in range(2 * GB // 16):
            pos_v[pl.ds(cnt + i * 16, 16)] = jnp.full((16,), EPAD - 1,
                                                      jnp.int32)
            loc_v[pl.ds(cnt + i * 16, 16)] = jnp.full((16,), W, jnp.int32)
        nb = (cnt + GB - 1) // GB

        def batch_body(b, _):
            pltpu.async_copy(
                msg_hbm.at[pos_v.at[pl.ds(b * GB, GB)]], rows0_v,
                sem0).wait()
            for j in range(GB // 16):
                locs = loc_v[pl.ds(b * GB + j * 16, 16)]
                for t in range(16):
                    l = locs[t]
                    e = j * 16 + t
                    for k in range(D // 16):
                        sl = pl.ds(k * 16, 16)
                        acc_v[l, sl] = acc_v[l, sl] + rows0_v[e, sl]
            return 0

        lax.fori_loop(0, nb, batch_body, 0)
        pltpu.sync_copy(acc_v.at[pl.ds(0, W)], out_hbm.at[c, pl.ds(lo, W)])

    return _sc_scatter_k


def _sc_scatter(msg, dst_r, zeros_nd):
    return _sc_scatter_kernel()(msg, dst_r, zeros_nd)


def _combine_body(x_ref, ea_ref, w_ref, r_ref, o_ref):
    ea_b = ea_ref[...].astype(jnp.bfloat16)
    x_b = x_ref[...].astype(jnp.bfloat16)
    ew = jnp.dot(ea_b, w_ref[...],
                 preferred_element_type=jnp.float32).astype(jnp.bfloat16)
    xrep = jnp.dot(x_b, r_ref[...], preferred_element_type=jnp.float32)
    z = xrep * ew.astype(jnp.float32)
    width = D * D
    while width > D:
        width //= 2
        z = z[:, :width] + z[:, width:2 * width]
    o_ref[...] = z


def _tc_combine(xg, ea, wcat, rep):
    return pl.pallas_call(
        _combine_body,
        grid=(EPAD // BE,),
        in_specs=[
            pl.BlockSpec((BE, D), lambda i: (i, 0)),
            pl.BlockSpec((BE, ED), lambda i: (i, 0)),
            pl.BlockSpec((ED, D * D), lambda i: (0, 0)),
            pl.BlockSpec((D, D * D), lambda i: (0, 0)),
        ],
        out_specs=pl.BlockSpec((BE, D), lambda i: (i, 0)),
        out_shape=jax.ShapeDtypeStruct((EPAD, D), jnp.float32),
    )(xg, ea, wcat, rep)


def _gru_body(p_ref, h_ref, w_ref, b_ref, o_ref):
    m = jnp.maximum(p_ref[0] + p_ref[1] + b_ref[6:7, :], 0.0)
    h = h_ref[...]

    def mm(a, k):
        return jnp.dot(a, w_ref[k], preferred_element_type=jnp.float32)

    r = jax.nn.sigmoid(mm(m, 0) + b_ref[0:1, :] + mm(h, 3) + b_ref[3:4, :])
    z = jax.nn.sigmoid(mm(m, 1) + b_ref[1:2, :] + mm(h, 4) + b_ref[4:5, :])
    n = jnp.tanh(mm(m, 2) + b_ref[2:3, :] + r * (mm(h, 5) + b_ref[5:6, :]))
    o_ref[...] = (1.0 - z) * n + z * h


def _tc_gru(parts, h, w6, b7):
    return pl.pallas_call(
        _gru_body,
        grid=(N // BN,),
        in_specs=[
            pl.BlockSpec((NC, BN, D), lambda i: (0, i, 0)),
            pl.BlockSpec((BN, D), lambda i: (i, 0)),
            pl.BlockSpec((6, D, D), lambda i: (0, 0, 0)),
            pl.BlockSpec((7, D), lambda i: (0, 0)),
        ],
        out_specs=pl.BlockSpec((BN, D), lambda i: (i, 0)),
        out_shape=jax.ShapeDtypeStruct((N, D), jnp.float32),
    )(parts, h, w6, b7)


def kernel(x, edge_index, edge_attr, We, b_conv, Wih, Whh, bih, bhh):
    src = edge_index[0]
    dst = edge_index[1]
    pad = EPAD - E
    src_r = jnp.concatenate([src, jnp.zeros((pad,), jnp.int32)]).reshape(
        NW, NCH, CH)
    dst_r = jnp.concatenate([dst, jnp.zeros((pad,), jnp.int32)]).reshape(
        NC, HALF)
    ea_p = jnp.concatenate(
        [edge_attr, jnp.zeros((pad, ED), jnp.float32)], axis=0)
    zeros_nd = jnp.zeros((W, D), jnp.float32)

    wcat_all = We.transpose(0, 2, 1).astype(jnp.bfloat16)
    rep = jnp.kron(jnp.eye(D, dtype=jnp.float32),
                   jnp.ones((1, D), jnp.float32)).astype(jnp.bfloat16)
    w6_all = jnp.stack([
        Wih[:, 0:D], Wih[:, D:2 * D], Wih[:, 2 * D:],
        Whh[:, 0:D], Whh[:, D:2 * D], Whh[:, 2 * D:],
    ], axis=1).transpose(0, 1, 3, 2)
    b7_all = jnp.stack([
        bih[:, 0:D], bih[:, D:2 * D], bih[:, 2 * D:],
        bhh[:, 0:D], bhh[:, D:2 * D], bhh[:, 2 * D:],
        b_conv,
    ], axis=1)

    def step(i, h):
        wcat = lax.dynamic_index_in_dim(wcat_all, i, 0, keepdims=False)
        w6 = lax.dynamic_index_in_dim(w6_all, i, 0, keepdims=False)
        b7 = lax.dynamic_index_in_dim(b7_all, i, 0, keepdims=False)
        xg = _sc_gather(h, src_r)
        msg = _tc_combine(xg, ea_p, wcat, rep)
        parts = _sc_scatter(msg, dst_r, zeros_nd)
        return _tc_gru(parts, h, w6, b7)

    return lax.fori_loop(0, S, step, x)


# speedup vs baseline: 1.0776x; 1.0078x over previous
import functools

import jax
import jax.numpy as jnp
from jax import lax
from jax.experimental import pallas as pl
from jax.experimental.pallas import tpu as pltpu
from jax.experimental.pallas import tpu_sc as plsc

S = 3
N = 10000
E = 30000
D = 64
ED = 16

NC = 2
NS = 16
NW = NC * NS

CH = 96
NCH = 10
EPW = CH * NCH
EPAD = EPW * NW
NPT = N // NS

NOUT = 10240
W = NOUT // NS
GB = 96
HALF = EPAD // NC
NG = HALF // 16

BE = 1024
BN = 1000

@functools.cache
def _sc_mesh():
    return plsc.VectorSubcoreMesh(
        core_axis_name="c", subcore_axis_name="s",
        num_cores=NC, num_subcores=NS)


@functools.cache
def _sc_gather_kernel():
    @functools.partial(
        pl.kernel,
        out_type=jax.ShapeDtypeStruct((EPAD, D), jnp.float32),
        mesh=_sc_mesh(),
        scratch_types=[
            pltpu.VMEM((NCH, CH), jnp.int32),
            pltpu.VMEM((EPW, D), jnp.float32),
            pltpu.SemaphoreType.DMA,
        ],
        compiler_params=pltpu.CompilerParams(use_tc_tiling_on_sc=False),
    )
    def _sc_gather(table_hbm, idx_hbm, out_hbm, idx_v, rows_v, sem):
        wid = lax.axis_index("s") * NC + lax.axis_index("c")
        pltpu.sync_copy(idx_hbm.at[wid], idx_v)
        descs = [
            pltpu.async_copy(
                table_hbm.at[idx_v.at[j]], rows_v.at[pl.ds(j * CH, CH)], sem)
            for j in range(NCH)
        ]
        for d in descs:
            d.wait()
        pltpu.sync_copy(rows_v, out_hbm.at[pl.ds(wid * EPW, EPW)])

    return _sc_gather


def _sc_gather(table, idx_r):
    return _sc_gather_kernel()(table, idx_r)


@functools.cache
def _sc_scatter_kernel():
    @functools.partial(
        pl.kernel,
        out_type=jax.ShapeDtypeStruct((NC, NOUT, D), jnp.float32),
        mesh=_sc_mesh(),
        scratch_types=[
            pltpu.VMEM((HALF,), jnp.int32),
            pltpu.VMEM((HALF + 2 * GB,), jnp.int32),
            pltpu.VMEM((HALF + 2 * GB,), jnp.int32),
            pltpu.VMEM((W + 8, D), jnp.float32),
            pltpu.VMEM((GB, D), jnp.float32),
            pltpu.SemaphoreType.DMA,
        ],
        compiler_params=pltpu.CompilerParams(
            use_tc_tiling_on_sc=False, needs_layout_passes=False),
    )
    def _sc_scatter_k(msg_hbm, dst_hbm, zeros_hbm, out_hbm,
                      dst_v, pos_v, loc_v, acc_v, rows0_v, sem0):
        c = lax.axis_index("c")
        s = lax.axis_index("s")
        lo = s * W
        pltpu.sync_copy(dst_hbm.at[c], dst_v)
        pltpu.sync_copy(zeros_hbm, acc_v.at[pl.ds(0, W)])
        iota = lax.iota(jnp.int32, 16)

        def scan_body(g, cur):
            v = dst_v[pl.ds(g * 16, 16)]
            l = v - lo
            m = (l >= 0) & (l < W)
            eid = c * HALF + g * 16 + iota
            plsc.store_compressed(pos_v.at[pl.ds(cur, 16)], eid, mask=m)
            plsc.store_compressed(loc_v.at[pl.ds(cur, 16)], l, mask=m)
            return cur + plsc.all_reduce_population_count(m)[0]

        cnt = lax.fori_loop(0, NG, scan_body, jnp.int32(0), unroll=8)
        for i in range(2 * GB // 16):
            pos_v[pl.ds(cnt + i * 16, 16)] = jnp.full((16,), EPAD - 1,
                                                      jnp.int32)
            loc_v[pl.ds(cnt + i * 16, 16)] = jnp.full((16,), W, jnp.int32)
        nb = (cnt + GB - 1) // GB

        def batch_body(b, _):
            pltpu.async_copy(
                msg_hbm.at[pos_v.at[pl.ds(b * GB, GB)]], rows0_v,
                sem0).wait()
            for j in range(GB // 16):
                locs = loc_v[pl.ds(b * GB + j * 16, 16)]
                for t in range(16):
                    l = locs[t]
                    e = j * 16 + t
                    for k in range(D // 16):
                        sl = pl.ds(k * 16, 16)
                        acc_v[l, sl] = acc_v[l, sl] + rows0_v[e, sl]
            return 0

        lax.fori_loop(0, nb, batch_body, 0)
        pltpu.sync_copy(acc_v.at[pl.ds(0, W)], out_hbm.at[c, pl.ds(lo, W)])

    return _sc_scatter_k


def _sc_scatter(msg, dst_r, zeros_nd):
    return _sc_scatter_kernel()(msg, dst_r, zeros_nd)


def _combine_body(x_ref, ea_ref, w_ref, r_ref, o_ref):
    ea_b = ea_ref[...].astype(jnp.bfloat16)
    x_b = x_ref[...].astype(jnp.bfloat16)
    ew = jnp.dot(ea_b, w_ref[...],
                 preferred_element_type=jnp.float32).astype(jnp.bfloat16)
    xrep = jnp.dot(x_b, r_ref[...], preferred_element_type=jnp.float32)
    z = xrep * ew.astype(jnp.float32)
    width = D * D
    while width > D:
        width //= 2
        z = z[:, :width] + z[:, width:2 * width]
    o_ref[...] = z


def _tc_combine(xg, ea, wcat, rep):
    return pl.pallas_call(
        _combine_body,
        grid=(EPAD // BE,),
        in_specs=[
            pl.BlockSpec((BE, D), lambda i: (i, 0)),
            pl.BlockSpec((BE, ED), lambda i: (i, 0)),
            pl.BlockSpec((ED, D * D), lambda i: (0, 0)),
            pl.BlockSpec((D, D * D), lambda i: (0, 0)),
        ],
        out_specs=pl.BlockSpec((BE, D), lambda i: (i, 0)),
        out_shape=jax.ShapeDtypeStruct((EPAD, D), jnp.float32),
    )(xg, ea, wcat, rep)


def _gru_body(p_ref, h_ref, w_ref, b_ref, o_ref):
    m = jnp.maximum(p_ref[0] + p_ref[1] + b_ref[6:7, :], 0.0)
    h = h_ref[...]

    def mm(a, k):
        return jnp.dot(a, w_ref[k], preferred_element_type=jnp.float32)

    r = jax.nn.sigmoid(mm(m, 0) + b_ref[0:1, :] + mm(h, 3) + b_ref[3:4, :])
    z = jax.nn.sigmoid(mm(m, 1) + b_ref[1:2, :] + mm(h, 4) + b_ref[4:5, :])
    n = jnp.tanh(mm(m, 2) + b_ref[2:3, :] + r * (mm(h, 5) + b_ref[5:6, :]))
    o_ref[...] = (1.0 - z) * n + z * h


def _tc_gru(parts, h, w6, b7):
    return pl.pallas_call(
        _gru_body,
        grid=(N // BN,),
        in_specs=[
            pl.BlockSpec((NC, BN, D), lambda i: (0, i, 0)),
            pl.BlockSpec((BN, D), lambda i: (i, 0)),
            pl.BlockSpec((6, D, D), lambda i: (0, 0, 0)),
            pl.BlockSpec((7, D), lambda i: (0, 0)),
        ],
        out_specs=pl.BlockSpec((BN, D), lambda i: (i, 0)),
        out_shape=jax.ShapeDtypeStruct((N, D), jnp.float32),
    )(parts, h, w6, b7)


def kernel(x, edge_index, edge_attr, We, b_conv, Wih, Whh, bih, bhh):
    src = edge_index[0]
    dst = edge_index[1]
    pad = EPAD - E
    src_r = jnp.concatenate([src, jnp.zeros((pad,), jnp.int32)]).reshape(
        NW, NCH, CH)
    dst_r = jnp.concatenate([dst, jnp.zeros((pad,), jnp.int32)]).reshape(
        NC, HALF)
    ea_p = jnp.concatenate(
        [edge_attr, jnp.zeros((pad, ED), jnp.float32)], axis=0)
    zeros_nd = jnp.zeros((W, D), jnp.float32)

    wcat_all = We.transpose(0, 2, 1).astype(jnp.bfloat16)
    rep = jnp.kron(jnp.eye(D, dtype=jnp.float32),
                   jnp.ones((1, D), jnp.float32)).astype(jnp.bfloat16)
    w6_all = jnp.stack([
        Wih[:, 0:D], Wih[:, D:2 * D], Wih[:, 2 * D:],
        Whh[:, 0:D], Whh[:, D:2 * D], Whh[:, 2 * D:],
    ], axis=1).transpose(0, 1, 3, 2)
    b7_all = jnp.stack([
        bih[:, 0:D], bih[:, D:2 * D], bih[:, 2 * D:],
        bhh[:, 0:D], bhh[:, D:2 * D], bhh[:, 2 * D:],
        b_conv,
    ], axis=1)

    def step(i, h):
        wcat = lax.dynamic_index_in_dim(wcat_all, i, 0, keepdims=False)
        w6 = lax.dynamic_index_in_dim(w6_all, i, 0, keepdims=False)
        b7 = lax.dynamic_index_in_dim(b7_all, i, 0, keepdims=False)
        xg = _sc_gather(h, src_r)
        msg = _tc_combine(xg, ea_p, wcat, rep)
        parts = _sc_scatter(msg, dst_r, zeros_nd)
        return _tc_gru(parts, h, w6, b7)

    return lax.fori_loop(0, S, step, x)
